# Initial kernel scaffold; baseline (speedup 1.0000x reference)
import jax
import jax.numpy as jnp
from jax.experimental import pallas as pl


def kernel(h_paper, h_author, edge_index_ap, edge_index_pa, W0_paper, W0_author, W1_paper, W1_author, W2_paper, W2_author, att0_paper, att0_author, att1_paper, att1_author, att2_paper, att2_author):
    raise NotImplementedError("write your pallas kernel here")


# speedup vs baseline: 6.9182x; 6.9182x over previous
import functools

import jax
import jax.numpy as jnp
from jax import lax
from jax.experimental import pallas as pl
from jax.experimental.pallas import tpu as pltpu
from jax.experimental.pallas import tpu_sc as plsc

N = 10000
E = 160000
TILES = 16
RPT = N // TILES

EPT2 = E // TILES
CH2 = 80
NCH2 = EPT2 // CH2

WORKERS = 2 * TILES
EPT1 = E // WORKERS
CH1 = 40
NCH1 = EPT1 // CH1


def _run_chunks(z_hbm, idx_src, idx_dst2, rows0, rows1, sem0, sem1,
                m_sh, ch, nch):
  def gather(j, rows, sem):
    pltpu.async_copy(z_hbm.at[idx_src.at[pl.ds(j * ch, ch)]], rows, sem)

  def gwait(j, rows, sem):
    pltpu.make_async_copy(z_hbm.at[idx_src.at[pl.ds(j * ch, ch)]], rows,
                          sem).wait()

  def scat(j, rows):
    pltpu.sync_copy(rows, m_sh.at[idx_dst2.at[j]], add=True)

  gather(0, rows0, sem0)

  def body(i, carry):
    j0 = 2 * i
    gwait(j0, rows0, sem0)
    gather(j0 + 1, rows1, sem1)
    scat(j0, rows0)
    gwait(j0 + 1, rows1, sem1)
    gather(j0 + 2, rows0, sem0)
    scat(j0 + 1, rows1)
    return carry

  lax.fori_loop(0, (nch - 1) // 2, body, 0)
  gwait(nch - 1, rows0, sem0)
  scat(nch - 1, rows0)


def _make_seg2(d):
  mesh = plsc.VectorSubcoreMesh(core_axis_name="c", subcore_axis_name="s")

  def one_relation(src_hbm, dst3_hbm, z_hbm, zero_hbm, out_hbm,
                   idx_src, idx_dst2, rows0, rows1, m_sh, sem0, sem1):
    s = lax.axis_index("s")
    rbase = s * RPT
    pltpu.sync_copy(zero_hbm.at[pl.ds(rbase, RPT)],
                    m_sh.at[pl.ds(rbase, RPT)])
    pltpu.sync_copy(src_hbm.at[pl.ds(s * EPT2, EPT2)], idx_src)
    pltpu.sync_copy(dst3_hbm.at[s], idx_dst2)
    plsc.subcore_barrier()
    _run_chunks(z_hbm, idx_src, idx_dst2, rows0, rows1, sem0, sem1,
                m_sh, CH2, NCH2)
    plsc.subcore_barrier()
    pltpu.sync_copy(m_sh.at[pl.ds(rbase, RPT)],
                    out_hbm.at[pl.ds(rbase, RPT)])

  @functools.partial(
      pl.kernel,
      out_type=(jax.ShapeDtypeStruct((N, d), jnp.float32),
                jax.ShapeDtypeStruct((N, d), jnp.float32)),
      mesh=mesh,
      compiler_params=pltpu.CompilerParams(use_tc_tiling_on_sc=False),
      scratch_types=[
          pltpu.VMEM((EPT2,), jnp.int32),
          pltpu.VMEM((NCH2, CH2), jnp.int32),
          pltpu.VMEM((CH2, d), jnp.float32),
          pltpu.VMEM((CH2, d), jnp.float32),
          pltpu.VMEM_SHARED((N, d), jnp.float32),
          pltpu.SemaphoreType.DMA,
          pltpu.SemaphoreType.DMA,
      ],
  )
  def seg2(zp, za, ap_src, ap_dst3, pa_src, pa_dst3, zero,
           mp_out, ma_out, idx_src, idx_dst2, rows0, rows1, m_sh,
           sem0, sem1):
    c = lax.axis_index("c")

    @pl.when(c == 0)
    def _():
      one_relation(ap_src, ap_dst3, za, zero, mp_out,
                   idx_src, idx_dst2, rows0, rows1, m_sh, sem0, sem1)

    @pl.when(c == 1)
    def _():
      one_relation(pa_src, pa_dst3, zp, zero, ma_out,
                   idx_src, idx_dst2, rows0, rows1, m_sh, sem0, sem1)

  return seg2


def _make_seg1(d):
  mesh = plsc.VectorSubcoreMesh(core_axis_name="c", subcore_axis_name="s")

  @functools.partial(
      pl.kernel,
      out_type=(jax.ShapeDtypeStruct((N, d), jnp.float32),
                jax.ShapeDtypeStruct((N, d), jnp.float32)),
      mesh=mesh,
      compiler_params=pltpu.CompilerParams(use_tc_tiling_on_sc=False),
      scratch_types=[
          pltpu.VMEM((EPT1,), jnp.int32),
          pltpu.VMEM((NCH1, CH1), jnp.int32),
          pltpu.VMEM((CH1, d), jnp.float32),
          pltpu.VMEM((CH1, d), jnp.float32),
          pltpu.VMEM_SHARED((N, d), jnp.float32),
          pltpu.SemaphoreType.DMA,
          pltpu.SemaphoreType.DMA,
      ],
  )
  def seg1(z, src, dst3, zero, p0_out, p1_out,
           idx_src, idx_dst2, rows0, rows1, m_sh, sem0, sem1):
    c = lax.axis_index("c")
    s = lax.axis_index("s")
    w = c * TILES + s
    rbase = s * RPT
    pltpu.sync_copy(zero.at[pl.ds(rbase, RPT)], m_sh.at[pl.ds(rbase, RPT)])
    pltpu.sync_copy(src.at[pl.ds(w * EPT1, EPT1)], idx_src)
    pltpu.sync_copy(dst3.at[w], idx_dst2)
    plsc.subcore_barrier()
    _run_chunks(z, idx_src, idx_dst2, rows0, rows1, sem0, sem1,
                m_sh, CH1, NCH1)
    plsc.subcore_barrier()

    @pl.when(c == 0)
    def _():
      pltpu.sync_copy(m_sh.at[pl.ds(rbase, RPT)],
                      p0_out.at[pl.ds(rbase, RPT)])

    @pl.when(c == 1)
    def _():
      pltpu.sync_copy(m_sh.at[pl.ds(rbase, RPT)],
                      p1_out.at[pl.ds(rbase, RPT)])

  return seg1


_seg16 = _make_seg2(16)
_seg8 = _make_seg2(8)
_seg1_16 = _make_seg1(16)

BR = 2000


def _linear2(hp, ha, Wp, Wa):
  din, dout = Wp.shape

  def body(hp_ref, ha_ref, wp_ref, wa_ref, zp_ref, za_ref):
    zp_ref[...] = jnp.dot(hp_ref[...], wp_ref[...],
                          preferred_element_type=jnp.float32)
    za_ref[...] = jnp.dot(ha_ref[...], wa_ref[...],
                          preferred_element_type=jnp.float32)

  return pl.pallas_call(
      body,
      grid=(N // BR,),
      in_specs=[pl.BlockSpec((BR, din), lambda i: (i, 0)),
                pl.BlockSpec((BR, din), lambda i: (i, 0)),
                pl.BlockSpec((din, dout), lambda i: (0, 0)),
                pl.BlockSpec((din, dout), lambda i: (0, 0))],
      out_specs=[pl.BlockSpec((BR, dout), lambda i: (i, 0)),
                 pl.BlockSpec((BR, dout), lambda i: (i, 0))],
      out_shape=[jax.ShapeDtypeStruct((N, dout), jnp.float32),
                 jax.ShapeDtypeStruct((N, dout), jnp.float32)],
  )(hp, ha, Wp, Wa)


def _att_mix(z, m, a):
  e0 = jnp.sum(z * a, axis=1, keepdims=True)
  e0 = jnp.where(e0 >= 0, e0, 0.2 * e0)
  e1 = jnp.sum(m * a, axis=1, keepdims=True)
  e1 = jnp.where(e1 >= 0, e1, 0.2 * e1)
  mx = jnp.maximum(e0, e1)
  x0 = jnp.exp(e0 - mx)
  x1 = jnp.exp(e1 - mx)
  return (x0 * z + x1 * m) / (x0 + x1)


def _elu(x):
  return jnp.where(x > 0, x, jnp.exp(jnp.minimum(x, 0.0)) - 1.0)


def _att_elu_linear2(zp, mp, ap, Wp, za, ma, aa, Wa):
  d = zp.shape[1]
  dout = Wp.shape[1]

  def body(zp_ref, mp_ref, ap_ref, wp_ref, za_ref, ma_ref, aa_ref, wa_ref,
           op_ref, oa_ref):
    hp = _elu(_att_mix(zp_ref[...], mp_ref[...], ap_ref[...]))
    ha = _elu(_att_mix(za_ref[...], ma_ref[...], aa_ref[...]))
    op_ref[...] = jnp.dot(hp, wp_ref[...], preferred_element_type=jnp.float32)
    oa_ref[...] = jnp.dot(ha, wa_ref[...], preferred_element_type=jnp.float32)

  return pl.pallas_call(
      body,
      grid=(N // BR,),
      in_specs=[pl.BlockSpec((BR, d), lambda i: (i, 0)),
                pl.BlockSpec((BR, d), lambda i: (i, 0)),
                pl.BlockSpec((1, d), lambda i: (0, 0)),
                pl.BlockSpec((d, dout), lambda i: (0, 0)),
                pl.BlockSpec((BR, d), lambda i: (i, 0)),
                pl.BlockSpec((BR, d), lambda i: (i, 0)),
                pl.BlockSpec((1, d), lambda i: (0, 0)),
                pl.BlockSpec((d, dout), lambda i: (0, 0))],
      out_specs=[pl.BlockSpec((BR, dout), lambda i: (i, 0)),
                 pl.BlockSpec((BR, dout), lambda i: (i, 0))],
      out_shape=[jax.ShapeDtypeStruct((N, dout), jnp.float32),
                 jax.ShapeDtypeStruct((N, dout), jnp.float32)],
  )(zp, mp, ap, Wp, za, ma, aa, Wa)


def _att_final(zp, m0, m1, ap):
  d = zp.shape[1]

  def body(zp_ref, m0_ref, m1_ref, ap_ref, o_ref):
    o_ref[...] = _att_mix(zp_ref[...], m0_ref[...] + m1_ref[...],
                          ap_ref[...])

  return pl.pallas_call(
      body,
      grid=(N // BR,),
      in_specs=[pl.BlockSpec((BR, d), lambda i: (i, 0)),
                pl.BlockSpec((BR, d), lambda i: (i, 0)),
                pl.BlockSpec((BR, d), lambda i: (i, 0)),
                pl.BlockSpec((1, d), lambda i: (0, 0))],
      out_specs=pl.BlockSpec((BR, d), lambda i: (i, 0)),
      out_shape=jax.ShapeDtypeStruct((N, d), jnp.float32),
  )(zp, m0, m1, ap)


def kernel(h_paper, h_author, edge_index_ap, edge_index_pa,
           W0_paper, W0_author, W1_paper, W1_author, W2_paper, W2_author,
           att0_paper, att0_author, att1_paper, att1_author,
           att2_paper, att2_author):
  ap_src = edge_index_ap[0].astype(jnp.int32)
  ap_dst = edge_index_ap[1].astype(jnp.int32)
  pa_src = edge_index_pa[0].astype(jnp.int32)
  pa_dst = edge_index_pa[1].astype(jnp.int32)
  ap_dst3 = ap_dst.reshape(TILES, NCH2, CH2)
  pa_dst3 = pa_dst.reshape(TILES, NCH2, CH2)
  ap_dst3w = ap_dst.reshape(WORKERS, NCH1, CH1)
  zero16 = jnp.zeros((N, 16), jnp.float32)
  zero8 = jnp.zeros((N, 8), jnp.float32)

  a0p = att0_paper.reshape(1, -1)
  a0a = att0_author.reshape(1, -1)
  a1p = att1_paper.reshape(1, -1)
  a1a = att1_author.reshape(1, -1)
  a2p = att2_paper.reshape(1, -1)

  z0p, z0a = _linear2(h_paper, h_author, W0_paper, W0_author)
  m0p, m0a = _seg16(z0p, z0a, ap_src, ap_dst3, pa_src, pa_dst3, zero16)
  z1p, z1a = _att_elu_linear2(z0p, m0p, a0p, W1_paper, z0a, m0a, a0a, W1_author)
  m1p, m1a = _seg8(z1p, z1a, ap_src, ap_dst3, pa_src, pa_dst3, zero8)
  z2p, z2a = _att_elu_linear2(z1p, m1p, a1p, W2_paper, z1a, m1a, a1a, W2_author)
  m2p0, m2p1 = _seg1_16(z2a, ap_src, ap_dst3w, zero16)
  return _att_final(z2p, m2p0, m2p1, a2p)


# speedup vs baseline: 9.8601x; 1.4252x over previous
import functools

import jax
import jax.numpy as jnp
from jax import lax
from jax.experimental import pallas as pl
from jax.experimental.pallas import tpu as pltpu
from jax.experimental.pallas import tpu_sc as plsc

N = 10000
E = 160000
TILES = 16
RPT = N // TILES
CH = 128
K = 4
E_PAD = 163840
NROWS = N + 8

EPT2 = E_PAD // TILES
NCH2 = EPT2 // CH

WORKERS = 2 * TILES
EPT1 = E_PAD // WORKERS
NCH1 = EPT1 // CH


def _run_chunks(z_hbm, idx_src, idx_dst2, rowsA, rowsB, semA, semB,
                m_sh, nch):
  ngrp = nch // K

  def fire(g, rows, sem):
    for b in range(K):
      j = g * K + b
      pltpu.async_copy(z_hbm.at[idx_src.at[pl.ds(j * CH, CH)]],
                       rows.at[b], sem)

  def drain(g, rows, sem):
    for b in range(K):
      j = g * K + b
      pltpu.make_async_copy(z_hbm.at[idx_src.at[pl.ds(j * CH, CH)]],
                            rows.at[b], sem).wait()

  def scat(g, rows):
    for b in range(K):
      j = g * K + b
      pltpu.sync_copy(rows.at[b], m_sh.at[idx_dst2.at[j]], add=True)

  fire(0, rowsA, semA)

  def body(i, carry):
    g0 = 2 * i
    drain(g0, rowsA, semA)
    fire(g0 + 1, rowsB, semB)
    scat(g0, rowsA)
    drain(g0 + 1, rowsB, semB)

    @pl.when(g0 + 2 < ngrp)
    def _():
      fire(g0 + 2, rowsA, semA)

    scat(g0 + 1, rowsB)
    return carry

  lax.fori_loop(0, ngrp // 2, body, 0)


def _make_seg2(d):
  mesh = plsc.VectorSubcoreMesh(core_axis_name="c", subcore_axis_name="s")

  def one_relation(src_hbm, dst3_hbm, z_hbm, zero_hbm, out_hbm,
                   idx_src, idx_dst2, rowsA, rowsB, m_sh, semA, semB):
    s = lax.axis_index("s")
    rbase = s * RPT
    pltpu.sync_copy(zero_hbm.at[pl.ds(rbase, RPT)],
                    m_sh.at[pl.ds(rbase, RPT)])
    pltpu.sync_copy(src_hbm.at[pl.ds(s * EPT2, EPT2)], idx_src)
    pltpu.sync_copy(dst3_hbm.at[s], idx_dst2)
    plsc.subcore_barrier()
    _run_chunks(z_hbm, idx_src, idx_dst2, rowsA, rowsB, semA, semB,
                m_sh, NCH2)
    plsc.subcore_barrier()
    pltpu.sync_copy(m_sh.at[pl.ds(rbase, RPT)],
                    out_hbm.at[pl.ds(rbase, RPT)])

  @functools.partial(
      pl.kernel,
      out_type=(jax.ShapeDtypeStruct((N, d), jnp.float32),
                jax.ShapeDtypeStruct((N, d), jnp.float32)),
      mesh=mesh,
      compiler_params=pltpu.CompilerParams(use_tc_tiling_on_sc=False),
      scratch_types=[
          pltpu.VMEM((EPT2,), jnp.int32),
          pltpu.VMEM((NCH2, CH), jnp.int32),
          pltpu.VMEM((K, CH, d), jnp.float32),
          pltpu.VMEM((K, CH, d), jnp.float32),
          pltpu.VMEM_SHARED((NROWS, d), jnp.float32),
          pltpu.SemaphoreType.DMA,
          pltpu.SemaphoreType.DMA,
      ],
  )
  def seg2(zp, za, ap_src, ap_dst3, pa_src, pa_dst3, zero,
           mp_out, ma_out, idx_src, idx_dst2, rowsA, rowsB, m_sh,
           semA, semB):
    c = lax.axis_index("c")

    @pl.when(c == 0)
    def _():
      one_relation(ap_src, ap_dst3, za, zero, mp_out,
                   idx_src, idx_dst2, rowsA, rowsB, m_sh, semA, semB)

    @pl.when(c == 1)
    def _():
      one_relation(pa_src, pa_dst3, zp, zero, ma_out,
                   idx_src, idx_dst2, rowsA, rowsB, m_sh, semA, semB)

  return seg2


def _make_seg1(d):
  mesh = plsc.VectorSubcoreMesh(core_axis_name="c", subcore_axis_name="s")

  @functools.partial(
      pl.kernel,
      out_type=(jax.ShapeDtypeStruct((N, d), jnp.float32),
                jax.ShapeDtypeStruct((N, d), jnp.float32)),
      mesh=mesh,
      compiler_params=pltpu.CompilerParams(use_tc_tiling_on_sc=False),
      scratch_types=[
          pltpu.VMEM((EPT1,), jnp.int32),
          pltpu.VMEM((NCH1, CH), jnp.int32),
          pltpu.VMEM((K, CH, d), jnp.float32),
          pltpu.VMEM((K, CH, d), jnp.float32),
          pltpu.VMEM_SHARED((NROWS, d), jnp.float32),
          pltpu.SemaphoreType.DMA,
          pltpu.SemaphoreType.DMA,
      ],
  )
  def seg1(z, src, dst3, zero, p0_out, p1_out,
           idx_src, idx_dst2, rowsA, rowsB, m_sh, semA, semB):
    c = lax.axis_index("c")
    s = lax.axis_index("s")
    w = c * TILES + s
    rbase = s * RPT
    pltpu.sync_copy(zero.at[pl.ds(rbase, RPT)], m_sh.at[pl.ds(rbase, RPT)])
    pltpu.sync_copy(src.at[pl.ds(w * EPT1, EPT1)], idx_src)
    pltpu.sync_copy(dst3.at[w], idx_dst2)
    plsc.subcore_barrier()
    _run_chunks(z, idx_src, idx_dst2, rowsA, rowsB, semA, semB,
                m_sh, NCH1)
    plsc.subcore_barrier()

    @pl.when(c == 0)
    def _():
      pltpu.sync_copy(m_sh.at[pl.ds(rbase, RPT)],
                      p0_out.at[pl.ds(rbase, RPT)])

    @pl.when(c == 1)
    def _():
      pltpu.sync_copy(m_sh.at[pl.ds(rbase, RPT)],
                      p1_out.at[pl.ds(rbase, RPT)])

  return seg1


_seg16 = _make_seg2(16)
_seg8 = _make_seg2(8)
_seg1_16 = _make_seg1(16)

BR = 2000


def _linear2(hp, ha, Wp, Wa):
  din, dout = Wp.shape

  def body(hp_ref, ha_ref, wp_ref, wa_ref, zp_ref, za_ref):
    zp_ref[...] = jnp.dot(hp_ref[...], wp_ref[...],
                          preferred_element_type=jnp.float32)
    za_ref[...] = jnp.dot(ha_ref[...], wa_ref[...],
                          preferred_element_type=jnp.float32)

  return pl.pallas_call(
      body,
      grid=(N // BR,),
      in_specs=[pl.BlockSpec((BR, din), lambda i: (i, 0)),
                pl.BlockSpec((BR, din), lambda i: (i, 0)),
                pl.BlockSpec((din, dout), lambda i: (0, 0)),
                pl.BlockSpec((din, dout), lambda i: (0, 0))],
      out_specs=[pl.BlockSpec((BR, dout), lambda i: (i, 0)),
                 pl.BlockSpec((BR, dout), lambda i: (i, 0))],
      out_shape=[jax.ShapeDtypeStruct((N, dout), jnp.float32),
                 jax.ShapeDtypeStruct((N, dout), jnp.float32)],
  )(hp, ha, Wp, Wa)


def _att_mix(z, m, a):
  e0 = jnp.sum(z * a, axis=1, keepdims=True)
  e0 = jnp.where(e0 >= 0, e0, 0.2 * e0)
  e1 = jnp.sum(m * a, axis=1, keepdims=True)
  e1 = jnp.where(e1 >= 0, e1, 0.2 * e1)
  mx = jnp.maximum(e0, e1)
  x0 = jnp.exp(e0 - mx)
  x1 = jnp.exp(e1 - mx)
  return (x0 * z + x1 * m) / (x0 + x1)


def _elu(x):
  return jnp.where(x > 0, x, jnp.exp(jnp.minimum(x, 0.0)) - 1.0)


def _att_elu_linear2(zp, mp, ap, Wp, za, ma, aa, Wa):
  d = zp.shape[1]
  dout = Wp.shape[1]

  def body(zp_ref, mp_ref, ap_ref, wp_ref, za_ref, ma_ref, aa_ref, wa_ref,
           op_ref, oa_ref):
    hp = _elu(_att_mix(zp_ref[...], mp_ref[...], ap_ref[...]))
    ha = _elu(_att_mix(za_ref[...], ma_ref[...], aa_ref[...]))
    op_ref[...] = jnp.dot(hp, wp_ref[...], preferred_element_type=jnp.float32)
    oa_ref[...] = jnp.dot(ha, wa_ref[...], preferred_element_type=jnp.float32)

  return pl.pallas_call(
      body,
      grid=(N // BR,),
      in_specs=[pl.BlockSpec((BR, d), lambda i: (i, 0)),
                pl.BlockSpec((BR, d), lambda i: (i, 0)),
                pl.BlockSpec((1, d), lambda i: (0, 0)),
                pl.BlockSpec((d, dout), lambda i: (0, 0)),
                pl.BlockSpec((BR, d), lambda i: (i, 0)),
                pl.BlockSpec((BR, d), lambda i: (i, 0)),
                pl.BlockSpec((1, d), lambda i: (0, 0)),
                pl.BlockSpec((d, dout), lambda i: (0, 0))],
      out_specs=[pl.BlockSpec((BR, dout), lambda i: (i, 0)),
                 pl.BlockSpec((BR, dout), lambda i: (i, 0))],
      out_shape=[jax.ShapeDtypeStruct((N, dout), jnp.float32),
                 jax.ShapeDtypeStruct((N, dout), jnp.float32)],
  )(zp, mp, ap, Wp, za, ma, aa, Wa)


def _att_final(zp, m0, m1, ap):
  d = zp.shape[1]

  def body(zp_ref, m0_ref, m1_ref, ap_ref, o_ref):
    o_ref[...] = _att_mix(zp_ref[...], m0_ref[...] + m1_ref[...],
                          ap_ref[...])

  return pl.pallas_call(
      body,
      grid=(N // BR,),
      in_specs=[pl.BlockSpec((BR, d), lambda i: (i, 0)),
                pl.BlockSpec((BR, d), lambda i: (i, 0)),
                pl.BlockSpec((BR, d), lambda i: (i, 0)),
                pl.BlockSpec((1, d), lambda i: (0, 0))],
      out_specs=pl.BlockSpec((BR, d), lambda i: (i, 0)),
      out_shape=jax.ShapeDtypeStruct((N, d), jnp.float32),
  )(zp, m0, m1, ap)


def kernel(h_paper, h_author, edge_index_ap, edge_index_pa,
           W0_paper, W0_author, W1_paper, W1_author, W2_paper, W2_author,
           att0_paper, att0_author, att1_paper, att1_author,
           att2_paper, att2_author):
  pad = E_PAD - E
  ap_src = jnp.concatenate(
      [edge_index_ap[0].astype(jnp.int32), jnp.zeros((pad,), jnp.int32)])
  ap_dst = jnp.concatenate(
      [edge_index_ap[1].astype(jnp.int32), jnp.full((pad,), N, jnp.int32)])
  pa_src = jnp.concatenate(
      [edge_index_pa[0].astype(jnp.int32), jnp.zeros((pad,), jnp.int32)])
  pa_dst = jnp.concatenate(
      [edge_index_pa[1].astype(jnp.int32), jnp.full((pad,), N, jnp.int32)])
  ap_dst3 = ap_dst.reshape(TILES, NCH2, CH)
  pa_dst3 = pa_dst.reshape(TILES, NCH2, CH)
  ap_dst3w = ap_dst.reshape(WORKERS, NCH1, CH)
  zero16 = jnp.zeros((N, 16), jnp.float32)
  zero8 = jnp.zeros((N, 8), jnp.float32)

  a0p = att0_paper.reshape(1, -1)
  a0a = att0_author.reshape(1, -1)
  a1p = att1_paper.reshape(1, -1)
  a1a = att1_author.reshape(1, -1)
  a2p = att2_paper.reshape(1, -1)

  z0p, z0a = _linear2(h_paper, h_author, W0_paper, W0_author)
  m0p, m0a = _seg16(z0p, z0a, ap_src, ap_dst3, pa_src, pa_dst3, zero16)
  z1p, z1a = _att_elu_linear2(z0p, m0p, a0p, W1_paper, z0a, m0a, a0a, W1_author)
  m1p, m1a = _seg8(z1p, z1a, ap_src, ap_dst3, pa_src, pa_dst3, zero8)
  z2p, z2a = _att_elu_linear2(z1p, m1p, a1p, W2_paper, z1a, m1a, a1a, W2_author)
  m2p0, m2p1 = _seg1_16(z2a, ap_src, ap_dst3w, zero16)
  return _att_final(z2p, m2p0, m2p1, a2p)


# speedup vs baseline: 13.2936x; 1.3482x over previous
import functools

import jax
import jax.numpy as jnp
from jax import lax
from jax.experimental import pallas as pl
from jax.experimental.pallas import tpu as pltpu
from jax.experimental.pallas import tpu_sc as plsc

N = 10000
E = 160000
TILES = 16
RPT = N // TILES
CH = 128
K = 4
E_PAD = 163840
NROWS = N + 8

EPT2 = E_PAD // TILES
NCH2 = EPT2 // CH

WORKERS = 2 * TILES
EPT1 = E_PAD // WORKERS
NCH1 = EPT1 // CH


def _run_chunks(z_hbm, idx_src, idx_dst2, rowsA, rowsB, semA, semB,
                m_sh, nch):
  ngrp = nch // K

  def fire(g, rows, sem):
    for b in range(K):
      j = g * K + b
      pltpu.async_copy(z_hbm.at[idx_src.at[pl.ds(j * CH, CH)]],
                       rows.at[b], sem)

  def drain(g, rows, sem):
    for b in range(K):
      j = g * K + b
      pltpu.make_async_copy(z_hbm.at[idx_src.at[pl.ds(j * CH, CH)]],
                            rows.at[b], sem).wait()

  def scat(g, rows):
    for b in range(K):
      j = g * K + b
      pltpu.sync_copy(rows.at[b], m_sh.at[idx_dst2.at[j]], add=True)

  fire(0, rowsA, semA)

  def body(i, carry):
    g0 = 2 * i
    drain(g0, rowsA, semA)
    fire(g0 + 1, rowsB, semB)
    scat(g0, rowsA)
    drain(g0 + 1, rowsB, semB)

    @pl.when(g0 + 2 < ngrp)
    def _():
      fire(g0 + 2, rowsA, semA)

    scat(g0 + 1, rowsB)
    return carry

  lax.fori_loop(0, ngrp // 2, body, 0)


def _make_seg2(d):
  mesh = plsc.VectorSubcoreMesh(core_axis_name="c", subcore_axis_name="s")

  def one_relation(src_hbm, dst3_hbm, z_hbm, zero_hbm, out_hbm,
                   idx_src, idx_dst2, rowsA, rowsB, z_sh, m_sh, semA, semB):
    s = lax.axis_index("s")
    rbase = s * RPT
    pltpu.sync_copy(zero_hbm.at[pl.ds(rbase, RPT)],
                    m_sh.at[pl.ds(rbase, RPT)])
    pltpu.sync_copy(z_hbm.at[pl.ds(rbase, RPT)],
                    z_sh.at[pl.ds(rbase, RPT)])
    pltpu.sync_copy(src_hbm.at[pl.ds(s * EPT2, EPT2)], idx_src)
    pltpu.sync_copy(dst3_hbm.at[s], idx_dst2)
    plsc.subcore_barrier()
    _run_chunks(z_sh, idx_src, idx_dst2, rowsA, rowsB, semA, semB,
                m_sh, NCH2)
    plsc.subcore_barrier()
    pltpu.sync_copy(m_sh.at[pl.ds(rbase, RPT)],
                    out_hbm.at[pl.ds(rbase, RPT)])

  @functools.partial(
      pl.kernel,
      out_type=(jax.ShapeDtypeStruct((N, d), jnp.float32),
                jax.ShapeDtypeStruct((N, d), jnp.float32)),
      mesh=mesh,
      compiler_params=pltpu.CompilerParams(use_tc_tiling_on_sc=False),
      scratch_types=[
          pltpu.VMEM((EPT2,), jnp.int32),
          pltpu.VMEM((NCH2, CH), jnp.int32),
          pltpu.VMEM((K, CH, d), jnp.float32),
          pltpu.VMEM((K, CH, d), jnp.float32),
          pltpu.VMEM_SHARED((N, d), jnp.float32),
          pltpu.VMEM_SHARED((NROWS, d), jnp.float32),
          pltpu.SemaphoreType.DMA,
          pltpu.SemaphoreType.DMA,
      ],
  )
  def seg2(zp, za, ap_src, ap_dst3, pa_src, pa_dst3, zero,
           mp_out, ma_out, idx_src, idx_dst2, rowsA, rowsB, z_sh, m_sh,
           semA, semB):
    c = lax.axis_index("c")

    @pl.when(c == 0)
    def _():
      one_relation(ap_src, ap_dst3, za, zero, mp_out,
                   idx_src, idx_dst2, rowsA, rowsB, z_sh, m_sh, semA, semB)

    @pl.when(c == 1)
    def _():
      one_relation(pa_src, pa_dst3, zp, zero, ma_out,
                   idx_src, idx_dst2, rowsA, rowsB, z_sh, m_sh, semA, semB)

  return seg2


def _make_seg1(d):
  mesh = plsc.VectorSubcoreMesh(core_axis_name="c", subcore_axis_name="s")

  @functools.partial(
      pl.kernel,
      out_type=(jax.ShapeDtypeStruct((N, d), jnp.float32),
                jax.ShapeDtypeStruct((N, d), jnp.float32)),
      mesh=mesh,
      compiler_params=pltpu.CompilerParams(use_tc_tiling_on_sc=False),
      scratch_types=[
          pltpu.VMEM((EPT1,), jnp.int32),
          pltpu.VMEM((NCH1, CH), jnp.int32),
          pltpu.VMEM((K, CH, d), jnp.float32),
          pltpu.VMEM((K, CH, d), jnp.float32),
          pltpu.VMEM_SHARED((N, d), jnp.float32),
          pltpu.VMEM_SHARED((NROWS, d), jnp.float32),
          pltpu.SemaphoreType.DMA,
          pltpu.SemaphoreType.DMA,
      ],
  )
  def seg1(z, src, dst3, zero, p0_out, p1_out,
           idx_src, idx_dst2, rowsA, rowsB, z_sh, m_sh, semA, semB):
    c = lax.axis_index("c")
    s = lax.axis_index("s")
    w = c * TILES + s
    rbase = s * RPT
    pltpu.sync_copy(zero.at[pl.ds(rbase, RPT)], m_sh.at[pl.ds(rbase, RPT)])
    pltpu.sync_copy(z.at[pl.ds(rbase, RPT)], z_sh.at[pl.ds(rbase, RPT)])
    pltpu.sync_copy(src.at[pl.ds(w * EPT1, EPT1)], idx_src)
    pltpu.sync_copy(dst3.at[w], idx_dst2)
    plsc.subcore_barrier()
    _run_chunks(z_sh, idx_src, idx_dst2, rowsA, rowsB, semA, semB,
                m_sh, NCH1)
    plsc.subcore_barrier()

    @pl.when(c == 0)
    def _():
      pltpu.sync_copy(m_sh.at[pl.ds(rbase, RPT)],
                      p0_out.at[pl.ds(rbase, RPT)])

    @pl.when(c == 1)
    def _():
      pltpu.sync_copy(m_sh.at[pl.ds(rbase, RPT)],
                      p1_out.at[pl.ds(rbase, RPT)])

  return seg1


_seg16 = _make_seg2(16)
_seg8 = _make_seg2(8)
_seg1_16 = _make_seg1(16)

BR = 2000


def _linear2(hp, ha, Wp, Wa):
  din, dout = Wp.shape

  def body(hp_ref, ha_ref, wp_ref, wa_ref, zp_ref, za_ref):
    zp_ref[...] = jnp.dot(hp_ref[...], wp_ref[...],
                          preferred_element_type=jnp.float32)
    za_ref[...] = jnp.dot(ha_ref[...], wa_ref[...],
                          preferred_element_type=jnp.float32)

  return pl.pallas_call(
      body,
      grid=(N // BR,),
      in_specs=[pl.BlockSpec((BR, din), lambda i: (i, 0)),
                pl.BlockSpec((BR, din), lambda i: (i, 0)),
                pl.BlockSpec((din, dout), lambda i: (0, 0)),
                pl.BlockSpec((din, dout), lambda i: (0, 0))],
      out_specs=[pl.BlockSpec((BR, dout), lambda i: (i, 0)),
                 pl.BlockSpec((BR, dout), lambda i: (i, 0))],
      out_shape=[jax.ShapeDtypeStruct((N, dout), jnp.float32),
                 jax.ShapeDtypeStruct((N, dout), jnp.float32)],
  )(hp, ha, Wp, Wa)


def _att_mix(z, m, a):
  e0 = jnp.sum(z * a, axis=1, keepdims=True)
  e0 = jnp.where(e0 >= 0, e0, 0.2 * e0)
  e1 = jnp.sum(m * a, axis=1, keepdims=True)
  e1 = jnp.where(e1 >= 0, e1, 0.2 * e1)
  mx = jnp.maximum(e0, e1)
  x0 = jnp.exp(e0 - mx)
  x1 = jnp.exp(e1 - mx)
  return (x0 * z + x1 * m) / (x0 + x1)


def _elu(x):
  return jnp.where(x > 0, x, jnp.exp(jnp.minimum(x, 0.0)) - 1.0)


def _att_elu_linear2(zp, mp, ap, Wp, za, ma, aa, Wa):
  d = zp.shape[1]
  dout = Wp.shape[1]

  def body(zp_ref, mp_ref, ap_ref, wp_ref, za_ref, ma_ref, aa_ref, wa_ref,
           op_ref, oa_ref):
    hp = _elu(_att_mix(zp_ref[...], mp_ref[...], ap_ref[...]))
    ha = _elu(_att_mix(za_ref[...], ma_ref[...], aa_ref[...]))
    op_ref[...] = jnp.dot(hp, wp_ref[...], preferred_element_type=jnp.float32)
    oa_ref[...] = jnp.dot(ha, wa_ref[...], preferred_element_type=jnp.float32)

  return pl.pallas_call(
      body,
      grid=(N // BR,),
      in_specs=[pl.BlockSpec((BR, d), lambda i: (i, 0)),
                pl.BlockSpec((BR, d), lambda i: (i, 0)),
                pl.BlockSpec((1, d), lambda i: (0, 0)),
                pl.BlockSpec((d, dout), lambda i: (0, 0)),
                pl.BlockSpec((BR, d), lambda i: (i, 0)),
                pl.BlockSpec((BR, d), lambda i: (i, 0)),
                pl.BlockSpec((1, d), lambda i: (0, 0)),
                pl.BlockSpec((d, dout), lambda i: (0, 0))],
      out_specs=[pl.BlockSpec((BR, dout), lambda i: (i, 0)),
                 pl.BlockSpec((BR, dout), lambda i: (i, 0))],
      out_shape=[jax.ShapeDtypeStruct((N, dout), jnp.float32),
                 jax.ShapeDtypeStruct((N, dout), jnp.float32)],
  )(zp, mp, ap, Wp, za, ma, aa, Wa)


def _att_final(zp, m0, m1, ap):
  d = zp.shape[1]

  def body(zp_ref, m0_ref, m1_ref, ap_ref, o_ref):
    o_ref[...] = _att_mix(zp_ref[...], m0_ref[...] + m1_ref[...],
                          ap_ref[...])

  return pl.pallas_call(
      body,
      grid=(N // BR,),
      in_specs=[pl.BlockSpec((BR, d), lambda i: (i, 0)),
                pl.BlockSpec((BR, d), lambda i: (i, 0)),
                pl.BlockSpec((BR, d), lambda i: (i, 0)),
                pl.BlockSpec((1, d), lambda i: (0, 0))],
      out_specs=pl.BlockSpec((BR, d), lambda i: (i, 0)),
      out_shape=jax.ShapeDtypeStruct((N, d), jnp.float32),
  )(zp, m0, m1, ap)


def kernel(h_paper, h_author, edge_index_ap, edge_index_pa,
           W0_paper, W0_author, W1_paper, W1_author, W2_paper, W2_author,
           att0_paper, att0_author, att1_paper, att1_author,
           att2_paper, att2_author):
  pad = E_PAD - E
  ap_src = jnp.concatenate(
      [edge_index_ap[0].astype(jnp.int32), jnp.zeros((pad,), jnp.int32)])
  ap_dst = jnp.concatenate(
      [edge_index_ap[1].astype(jnp.int32), jnp.full((pad,), N, jnp.int32)])
  pa_src = jnp.concatenate(
      [edge_index_pa[0].astype(jnp.int32), jnp.zeros((pad,), jnp.int32)])
  pa_dst = jnp.concatenate(
      [edge_index_pa[1].astype(jnp.int32), jnp.full((pad,), N, jnp.int32)])
  ap_dst3 = ap_dst.reshape(TILES, NCH2, CH)
  pa_dst3 = pa_dst.reshape(TILES, NCH2, CH)
  ap_dst3w = ap_dst.reshape(WORKERS, NCH1, CH)
  zero16 = jnp.zeros((N, 16), jnp.float32)
  zero8 = jnp.zeros((N, 8), jnp.float32)

  a0p = att0_paper.reshape(1, -1)
  a0a = att0_author.reshape(1, -1)
  a1p = att1_paper.reshape(1, -1)
  a1a = att1_author.reshape(1, -1)
  a2p = att2_paper.reshape(1, -1)

  z0p, z0a = _linear2(h_paper, h_author, W0_paper, W0_author)
  m0p, m0a = _seg16(z0p, z0a, ap_src, ap_dst3, pa_src, pa_dst3, zero16)
  z1p, z1a = _att_elu_linear2(z0p, m0p, a0p, W1_paper, z0a, m0a, a0a, W1_author)
  m1p, m1a = _seg8(z1p, z1a, ap_src, ap_dst3, pa_src, pa_dst3, zero8)
  z2p, z2a = _att_elu_linear2(z1p, m1p, a1p, W2_paper, z1a, m1a, a1a, W2_author)
  m2p0, m2p1 = _seg1_16(z2a, ap_src, ap_dst3w, zero16)
  return _att_final(z2p, m2p0, m2p1, a2p)


# speedup vs baseline: 13.8885x; 1.0448x over previous
import functools

import jax
import jax.numpy as jnp
from jax import lax
from jax.experimental import pallas as pl
from jax.experimental.pallas import tpu as pltpu
from jax.experimental.pallas import tpu_sc as plsc

N = 10000
E = 160000
TILES = 16
RPT = N // TILES
CH = 128
K = 4
E_PAD = 163840
NROWS = N + 8

EPT2 = E_PAD // TILES
NCH2 = EPT2 // CH

WORKERS = 2 * TILES
EPT1 = E_PAD // WORKERS
NCH1 = EPT1 // CH


def _run_chunks(z_hbm, idx_src, idx_dst2, rowsA, rowsB, semA, semB,
                ssem, m_sh, nch):
  ngrp = nch // K

  def fire(g, rows, sem):
    for b in range(K):
      j = g * K + b
      pltpu.async_copy(z_hbm.at[idx_src.at[pl.ds(j * CH, CH)]],
                       rows.at[b], sem)

  def drain(g, rows, sem):
    for b in range(K):
      j = g * K + b
      pltpu.make_async_copy(z_hbm.at[idx_src.at[pl.ds(j * CH, CH)]],
                            rows.at[b], sem).wait()

  def scat_fire(g, rows):
    for b in range(K):
      j = g * K + b
      pltpu.async_copy(rows.at[b], m_sh.at[idx_dst2.at[j]], ssem, add=True)

  def scat_drain(g, rows):
    for b in range(K):
      j = g * K + b
      pltpu.make_async_copy(rows.at[b], m_sh.at[idx_dst2.at[j]],
                            ssem).wait()

  fire(0, rowsA, semA)

  def body(i, carry):
    g0 = 2 * i
    drain(g0, rowsA, semA)
    scat_fire(g0, rowsA)
    fire(g0 + 1, rowsB, semB)
    scat_drain(g0, rowsA)

    @pl.when(g0 + 2 < ngrp)
    def _():
      fire(g0 + 2, rowsA, semA)

    drain(g0 + 1, rowsB, semB)
    scat_fire(g0 + 1, rowsB)
    scat_drain(g0 + 1, rowsB)
    return carry

  lax.fori_loop(0, ngrp // 2, body, 0)


def _make_seg2(d):
  mesh = plsc.VectorSubcoreMesh(core_axis_name="c", subcore_axis_name="s")

  def one_relation(src_hbm, dst3_hbm, z_hbm, zero_hbm, out_hbm,
                   idx_src, idx_dst2, rowsA, rowsB, z_sh, m_sh,
                   semA, semB, ssem):
    s = lax.axis_index("s")
    rbase = s * RPT
    c0 = pltpu.async_copy(zero_hbm.at[pl.ds(rbase, RPT)],
                          m_sh.at[pl.ds(rbase, RPT)], semA)
    c1 = pltpu.async_copy(z_hbm.at[pl.ds(rbase, RPT)],
                          z_sh.at[pl.ds(rbase, RPT)], semB)
    c2 = pltpu.async_copy(src_hbm.at[pl.ds(s * EPT2, EPT2)], idx_src, ssem)
    c3 = pltpu.async_copy(dst3_hbm.at[s], idx_dst2, ssem)
    c0.wait()
    c1.wait()
    c2.wait()
    c3.wait()
    plsc.subcore_barrier()
    _run_chunks(z_sh, idx_src, idx_dst2, rowsA, rowsB, semA, semB,
                ssem, m_sh, NCH2)
    plsc.subcore_barrier()
    pltpu.sync_copy(m_sh.at[pl.ds(rbase, RPT)],
                    out_hbm.at[pl.ds(rbase, RPT)])

  @functools.partial(
      pl.kernel,
      out_type=(jax.ShapeDtypeStruct((N, d), jnp.float32),
                jax.ShapeDtypeStruct((N, d), jnp.float32)),
      mesh=mesh,
      compiler_params=pltpu.CompilerParams(use_tc_tiling_on_sc=False),
      scratch_types=[
          pltpu.VMEM((EPT2,), jnp.int32),
          pltpu.VMEM((NCH2, CH), jnp.int32),
          pltpu.VMEM((K, CH, d), jnp.float32),
          pltpu.VMEM((K, CH, d), jnp.float32),
          pltpu.VMEM_SHARED((N, d), jnp.float32),
          pltpu.VMEM_SHARED((NROWS, d), jnp.float32),
          pltpu.SemaphoreType.DMA,
          pltpu.SemaphoreType.DMA,
          pltpu.SemaphoreType.DMA,
      ],
  )
  def seg2(zp, za, ap_src, ap_dst3, pa_src, pa_dst3, zero,
           mp_out, ma_out, idx_src, idx_dst2, rowsA, rowsB, z_sh, m_sh,
           semA, semB, ssem):
    c = lax.axis_index("c")

    @pl.when(c == 0)
    def _():
      one_relation(ap_src, ap_dst3, za, zero, mp_out,
                   idx_src, idx_dst2, rowsA, rowsB, z_sh, m_sh,
                   semA, semB, ssem)

    @pl.when(c == 1)
    def _():
      one_relation(pa_src, pa_dst3, zp, zero, ma_out,
                   idx_src, idx_dst2, rowsA, rowsB, z_sh, m_sh,
                   semA, semB, ssem)

  return seg2


def _make_seg1(d):
  mesh = plsc.VectorSubcoreMesh(core_axis_name="c", subcore_axis_name="s")

  @functools.partial(
      pl.kernel,
      out_type=(jax.ShapeDtypeStruct((N, d), jnp.float32),
                jax.ShapeDtypeStruct((N, d), jnp.float32)),
      mesh=mesh,
      compiler_params=pltpu.CompilerParams(use_tc_tiling_on_sc=False),
      scratch_types=[
          pltpu.VMEM((EPT1,), jnp.int32),
          pltpu.VMEM((NCH1, CH), jnp.int32),
          pltpu.VMEM((K, CH, d), jnp.float32),
          pltpu.VMEM((K, CH, d), jnp.float32),
          pltpu.VMEM_SHARED((N, d), jnp.float32),
          pltpu.VMEM_SHARED((NROWS, d), jnp.float32),
          pltpu.SemaphoreType.DMA,
          pltpu.SemaphoreType.DMA,
          pltpu.SemaphoreType.DMA,
      ],
  )
  def seg1(z, src, dst3, zero, p0_out, p1_out,
           idx_src, idx_dst2, rowsA, rowsB, z_sh, m_sh, semA, semB, ssem):
    c = lax.axis_index("c")
    s = lax.axis_index("s")
    w = c * TILES + s
    rbase = s * RPT
    c0 = pltpu.async_copy(zero.at[pl.ds(rbase, RPT)],
                          m_sh.at[pl.ds(rbase, RPT)], semA)
    c1 = pltpu.async_copy(z.at[pl.ds(rbase, RPT)],
                          z_sh.at[pl.ds(rbase, RPT)], semB)
    c2 = pltpu.async_copy(src.at[pl.ds(w * EPT1, EPT1)], idx_src, ssem)
    c3 = pltpu.async_copy(dst3.at[w], idx_dst2, ssem)
    c0.wait()
    c1.wait()
    c2.wait()
    c3.wait()
    plsc.subcore_barrier()
    _run_chunks(z_sh, idx_src, idx_dst2, rowsA, rowsB, semA, semB,
                ssem, m_sh, NCH1)
    plsc.subcore_barrier()

    @pl.when(c == 0)
    def _():
      pltpu.sync_copy(m_sh.at[pl.ds(rbase, RPT)],
                      p0_out.at[pl.ds(rbase, RPT)])

    @pl.when(c == 1)
    def _():
      pltpu.sync_copy(m_sh.at[pl.ds(rbase, RPT)],
                      p1_out.at[pl.ds(rbase, RPT)])

  return seg1


_seg16 = _make_seg2(16)
_seg8 = _make_seg2(8)
_seg1_16 = _make_seg1(16)

BR = 2000


def _linear2(hp, ha, Wp, Wa):
  din, dout = Wp.shape

  def body(hp_ref, ha_ref, wp_ref, wa_ref, zp_ref, za_ref):
    zp_ref[...] = jnp.dot(hp_ref[...], wp_ref[...],
                          preferred_element_type=jnp.float32)
    za_ref[...] = jnp.dot(ha_ref[...], wa_ref[...],
                          preferred_element_type=jnp.float32)

  return pl.pallas_call(
      body,
      grid=(N // BR,),
      in_specs=[pl.BlockSpec((BR, din), lambda i: (i, 0)),
                pl.BlockSpec((BR, din), lambda i: (i, 0)),
                pl.BlockSpec((din, dout), lambda i: (0, 0)),
                pl.BlockSpec((din, dout), lambda i: (0, 0))],
      out_specs=[pl.BlockSpec((BR, dout), lambda i: (i, 0)),
                 pl.BlockSpec((BR, dout), lambda i: (i, 0))],
      out_shape=[jax.ShapeDtypeStruct((N, dout), jnp.float32),
                 jax.ShapeDtypeStruct((N, dout), jnp.float32)],
  )(hp, ha, Wp, Wa)


def _att_mix(z, m, a):
  e0 = jnp.sum(z * a, axis=1, keepdims=True)
  e0 = jnp.where(e0 >= 0, e0, 0.2 * e0)
  e1 = jnp.sum(m * a, axis=1, keepdims=True)
  e1 = jnp.where(e1 >= 0, e1, 0.2 * e1)
  mx = jnp.maximum(e0, e1)
  x0 = jnp.exp(e0 - mx)
  x1 = jnp.exp(e1 - mx)
  return (x0 * z + x1 * m) / (x0 + x1)


def _elu(x):
  return jnp.where(x > 0, x, jnp.exp(jnp.minimum(x, 0.0)) - 1.0)


def _att_elu_linear2(zp, mp, ap, Wp, za, ma, aa, Wa):
  d = zp.shape[1]
  dout = Wp.shape[1]

  def body(zp_ref, mp_ref, ap_ref, wp_ref, za_ref, ma_ref, aa_ref, wa_ref,
           op_ref, oa_ref):
    hp = _elu(_att_mix(zp_ref[...], mp_ref[...], ap_ref[...]))
    ha = _elu(_att_mix(za_ref[...], ma_ref[...], aa_ref[...]))
    op_ref[...] = jnp.dot(hp, wp_ref[...], preferred_element_type=jnp.float32)
    oa_ref[...] = jnp.dot(ha, wa_ref[...], preferred_element_type=jnp.float32)

  return pl.pallas_call(
      body,
      grid=(N // BR,),
      in_specs=[pl.BlockSpec((BR, d), lambda i: (i, 0)),
                pl.BlockSpec((BR, d), lambda i: (i, 0)),
                pl.BlockSpec((1, d), lambda i: (0, 0)),
                pl.BlockSpec((d, dout), lambda i: (0, 0)),
                pl.BlockSpec((BR, d), lambda i: (i, 0)),
                pl.BlockSpec((BR, d), lambda i: (i, 0)),
                pl.BlockSpec((1, d), lambda i: (0, 0)),
                pl.BlockSpec((d, dout), lambda i: (0, 0))],
      out_specs=[pl.BlockSpec((BR, dout), lambda i: (i, 0)),
                 pl.BlockSpec((BR, dout), lambda i: (i, 0))],
      out_shape=[jax.ShapeDtypeStruct((N, dout), jnp.float32),
                 jax.ShapeDtypeStruct((N, dout), jnp.float32)],
  )(zp, mp, ap, Wp, za, ma, aa, Wa)


def _att_final(zp, m0, m1, ap):
  d = zp.shape[1]

  def body(zp_ref, m0_ref, m1_ref, ap_ref, o_ref):
    o_ref[...] = _att_mix(zp_ref[...], m0_ref[...] + m1_ref[...],
                          ap_ref[...])

  return pl.pallas_call(
      body,
      grid=(N // BR,),
      in_specs=[pl.BlockSpec((BR, d), lambda i: (i, 0)),
                pl.BlockSpec((BR, d), lambda i: (i, 0)),
                pl.BlockSpec((BR, d), lambda i: (i, 0)),
                pl.BlockSpec((1, d), lambda i: (0, 0))],
      out_specs=pl.BlockSpec((BR, d), lambda i: (i, 0)),
      out_shape=jax.ShapeDtypeStruct((N, d), jnp.float32),
  )(zp, m0, m1, ap)


def kernel(h_paper, h_author, edge_index_ap, edge_index_pa,
           W0_paper, W0_author, W1_paper, W1_author, W2_paper, W2_author,
           att0_paper, att0_author, att1_paper, att1_author,
           att2_paper, att2_author):
  pad = E_PAD - E
  ap_src = jnp.concatenate(
      [edge_index_ap[0].astype(jnp.int32), jnp.zeros((pad,), jnp.int32)])
  ap_dst = jnp.concatenate(
      [edge_index_ap[1].astype(jnp.int32), jnp.full((pad,), N, jnp.int32)])
  pa_src = jnp.concatenate(
      [edge_index_pa[0].astype(jnp.int32), jnp.zeros((pad,), jnp.int32)])
  pa_dst = jnp.concatenate(
      [edge_index_pa[1].astype(jnp.int32), jnp.full((pad,), N, jnp.int32)])
  ap_dst3 = ap_dst.reshape(TILES, NCH2, CH)
  pa_dst3 = pa_dst.reshape(TILES, NCH2, CH)
  ap_dst3w = ap_dst.reshape(WORKERS, NCH1, CH)
  zero16 = jnp.zeros((N, 16), jnp.float32)
  zero8 = jnp.zeros((N, 8), jnp.float32)

  a0p = att0_paper.reshape(1, -1)
  a0a = att0_author.reshape(1, -1)
  a1p = att1_paper.reshape(1, -1)
  a1a = att1_author.reshape(1, -1)
  a2p = att2_paper.reshape(1, -1)

  z0p, z0a = _linear2(h_paper, h_author, W0_paper, W0_author)
  m0p, m0a = _seg16(z0p, z0a, ap_src, ap_dst3, pa_src, pa_dst3, zero16)
  z1p, z1a = _att_elu_linear2(z0p, m0p, a0p, W1_paper, z0a, m0a, a0a, W1_author)
  m1p, m1a = _seg8(z1p, z1a, ap_src, ap_dst3, pa_src, pa_dst3, zero8)
  z2p, z2a = _att_elu_linear2(z1p, m1p, a1p, W2_paper, z1a, m1a, a1a, W2_author)
  m2p0, m2p1 = _seg1_16(z2a, ap_src, ap_dst3w, zero16)
  return _att_final(z2p, m2p0, m2p1, a2p)
